# baseline (speedup 1.0000x reference)
import functools
import math

import jax
import jax.numpy as jnp
from jax import lax
from jax.experimental import pallas as pl
from jax.experimental.pallas import tpu as pltpu
from jax.experimental.pallas import tpu_sc as plsc

D_MODEL = 512
SCALE = math.sqrt(D_MODEL)

_info = plsc.get_sparse_core_info()
NC = _info.num_cores
NS = _info.num_subcores
NW = NC * NS

B = 4 * 2048
B_PER_W = B // NW
CHUNK = 64
NBUF = 3
NCHUNK = B_PER_W // CHUNK
VECS_PER_ROW = D_MODEL // 16

_mesh = plsc.VectorSubcoreMesh(core_axis_name="c", subcore_axis_name="s")


@functools.partial(
    pl.kernel,
    mesh=_mesh,
    out_type=jax.ShapeDtypeStruct((B, D_MODEL), jnp.float32),
    scratch_types=[
        pltpu.VMEM((B_PER_W,), jnp.int32),
        pltpu.VMEM((NBUF, CHUNK, D_MODEL), jnp.float32),
    ]
    + [pltpu.SemaphoreType.DMA] * (2 * NBUF),
)
def _emb_lookup(x_hbm, table_hbm, out_hbm, idx_v, rows_v, *sems):
    wid = lax.axis_index("s") * NC + lax.axis_index("c")
    base = wid * B_PER_W
    gsem = sems[:NBUF]
    ssem = sems[NBUF:]
    row = wid // (2048 // B_PER_W)
    col = (wid % (2048 // B_PER_W)) * B_PER_W
    pltpu.sync_copy(x_hbm.at[row, pl.ds(col, B_PER_W)], idx_v)

    def gather(c):
        return pltpu.async_copy(
            table_hbm.at[idx_v.at[pl.ds(c * CHUNK, CHUNK)]],
            rows_v.at[c % NBUF],
            gsem[c % NBUF],
        )

    def store(c):
        return pltpu.async_copy(
            rows_v.at[c % NBUF],
            out_hbm.at[pl.ds(base + c * CHUNK, CHUNK)],
            ssem[c % NBUF],
        )

    lookahead = NBUF - 1
    gathers = [gather(c) for c in range(lookahead)] + [None] * (NCHUNK - lookahead)
    stores = [None] * NCHUNK
    store_waited = [False] * NCHUNK
    for c in range(NCHUNK):
        gathers[c].wait()
        g = c + lookahead
        if g < NCHUNK:
            prev = g - NBUF
            if prev >= 0:
                stores[prev].wait()
                store_waited[prev] = True
            gathers[g] = gather(g)

        @plsc.parallel_loop(0, CHUNK, unroll=1)
        def _scale_row(i):
            buf = rows_v.at[c % NBUF]
            for j in range(VECS_PER_ROW):
                buf[i, pl.ds(j * 16, 16)] = buf[i, pl.ds(j * 16, 16)] * SCALE

        stores[c] = store(c)
    for c in range(NCHUNK):
        if not store_waited[c]:
            stores[c].wait()


@jax.jit
def kernel(x, table):
    out = _emb_lookup(x.astype(jnp.int32), table)
    return out.reshape(x.shape + (D_MODEL,))


# speedup vs baseline: 1.0079x; 1.0079x over previous
import functools
import math

import jax
import jax.numpy as jnp
from jax import lax
from jax.experimental import pallas as pl
from jax.experimental.pallas import tpu as pltpu
from jax.experimental.pallas import tpu_sc as plsc

D_MODEL = 512
SCALE = math.sqrt(D_MODEL)

_info = plsc.get_sparse_core_info()
NC = _info.num_cores
NS = _info.num_subcores
NW = NC * NS

B = 4 * 2048
B_PER_W = B // NW
CHUNK = 64
NBUF = 3
CHUNKS = (32, 64, 64, 64, 32)
assert sum(CHUNKS) == B_PER_W
OFFS = tuple(sum(CHUNKS[:i]) for i in range(len(CHUNKS)))
NCHUNK = len(CHUNKS)
VECS_PER_ROW = D_MODEL // 16

_mesh = plsc.VectorSubcoreMesh(core_axis_name="c", subcore_axis_name="s")


@functools.partial(
    pl.kernel,
    mesh=_mesh,
    out_type=jax.ShapeDtypeStruct((B, D_MODEL), jnp.float32),
    scratch_types=[
        pltpu.VMEM((B_PER_W,), jnp.int32),
        pltpu.VMEM((NBUF, CHUNK, D_MODEL), jnp.float32),
    ]
    + [pltpu.SemaphoreType.DMA] * (2 * NBUF),
)
def _emb_lookup(x_hbm, table_hbm, out_hbm, idx_v, rows_v, *sems):
    wid = lax.axis_index("s") * NC + lax.axis_index("c")
    base = wid * B_PER_W
    gsem = sems[:NBUF]
    ssem = sems[NBUF:]
    row = wid // (2048 // B_PER_W)
    col = (wid % (2048 // B_PER_W)) * B_PER_W
    pltpu.sync_copy(x_hbm.at[row, pl.ds(col, B_PER_W)], idx_v)

    def gather(c):
        sz = CHUNKS[c]
        return pltpu.async_copy(
            table_hbm.at[idx_v.at[pl.ds(OFFS[c], sz)]],
            rows_v.at[c % NBUF, pl.ds(0, sz)],
            gsem[c % NBUF],
        )

    def store(c):
        sz = CHUNKS[c]
        return pltpu.async_copy(
            rows_v.at[c % NBUF, pl.ds(0, sz)],
            out_hbm.at[pl.ds(base + OFFS[c], sz)],
            ssem[c % NBUF],
        )

    lookahead = NBUF - 1
    gathers = [gather(c) for c in range(lookahead)] + [None] * (NCHUNK - lookahead)
    stores = [None] * NCHUNK
    store_waited = [False] * NCHUNK
    for c in range(NCHUNK):
        gathers[c].wait()
        g = c + lookahead
        if g < NCHUNK:
            prev = g - NBUF
            if prev >= 0:
                stores[prev].wait()
                store_waited[prev] = True
            gathers[g] = gather(g)

        @plsc.parallel_loop(0, CHUNKS[c], unroll=1)
        def _scale_row(i):
            buf = rows_v.at[c % NBUF]
            for j in range(VECS_PER_ROW):
                buf[i, pl.ds(j * 16, 16)] = buf[i, pl.ds(j * 16, 16)] * SCALE

        stores[c] = store(c)
    for c in range(NCHUNK):
        if not store_waited[c]:
            stores[c].wait()


@jax.jit
def kernel(x, table):
    out = _emb_lookup(x.astype(jnp.int32), table)
    return out.reshape(x.shape + (D_MODEL,))
